# baseline (speedup 1.0000x reference)
import functools
import jax
import jax.numpy as jnp
from jax import lax
from jax.experimental import pallas as pl
from jax.experimental.pallas import tpu as pltpu
from jax.experimental.pallas import tpu_sc as plsc

VOCAB = 1000000
EMBED_DIM = 64
BATCH = 4096
HIST = 50
HPAD = 64

NUM_CORES = 2
NUM_SUBCORES = 16
NW = NUM_CORES * NUM_SUBCORES
BBLK = BATCH // NW

TBLK = 8192
TGRID = (VOCAB + TBLK - 1) // TBLK

_mesh = plsc.VectorSubcoreMesh(
    core_axis_name="c", subcore_axis_name="s",
    num_cores=NUM_CORES, num_subcores=NUM_SUBCORES,
)


def _pack_body(tt_ref, out_ref):
    y = tt_ref[...].T.reshape(TBLK // 2, 2, EMBED_DIM)
    out_ref[...] = jnp.concatenate([y[:, 0, :], y[:, 1, :]], axis=-1)


def _pack_table(tableT):
    return pl.pallas_call(
        _pack_body,
        grid=(TGRID,),
        in_specs=[pl.BlockSpec((EMBED_DIM, TBLK), lambda b: (0, b))],
        out_specs=pl.BlockSpec((TBLK // 2, 2 * EMBED_DIM), lambda b: (b, 0)),
        out_shape=jax.ShapeDtypeStruct((VOCAB // 2, 2 * EMBED_DIM),
                                       jnp.float32),
    )(tableT)


@functools.partial(
    pl.kernel,
    mesh=_mesh,
    out_type=jax.ShapeDtypeStruct((HIST, 8, NW, 8, BBLK), jnp.float32),
    scratch_types=[
        pltpu.VMEM((HPAD, BBLK), jnp.int32),
        pltpu.VMEM((HPAD, BBLK), jnp.int32),
        [pltpu.VMEM((BBLK, 128), jnp.float32) for _ in range(2)],
        [pltpu.VMEM((8, 8, BBLK), jnp.float32) for _ in range(2)],
        [pltpu.SemaphoreType.DMA for _ in range(2)],
        [pltpu.SemaphoreType.DMA for _ in range(2)],
    ],
    compiler_params=pltpu.CompilerParams(use_tc_tiling_on_sc=True,
                                         needs_layout_passes=False),
)
def _embed_gather(idxh_hbm, colb_hbm, table2_hbm, out_hbm,
                  idx_v, col_v, abuf, bbuf, gsem, osem):
    wid = lax.axis_index("s") * NUM_CORES + lax.axis_index("c")

    pltpu.sync_copy(idxh_hbm.at[wid], idx_v)
    pltpu.sync_copy(colb_hbm.at[wid], col_v)

    iotas = [lax.iota(jnp.int32, 16) + 16 * k for k in range(8)]

    pltpu.async_copy(table2_hbm.at[idx_v.at[0]], abuf[0], gsem[0])

    def step(h, pb):
        @pl.when(h + 1 < HIST)
        def _fire_gather():
            pltpu.async_copy(table2_hbm.at[idx_v.at[h + 1]], abuf[1 - pb],
                             gsem[1 - pb])

        pltpu.make_async_copy(table2_hbm.at[idx_v.at[h]], abuf[pb],
                              gsem[pb]).wait()

        @pl.when(h >= 2)
        def _drain_out():
            pltpu.make_async_copy(bbuf[pb], out_hbm.at[h - 2, :, wid],
                                  osem[pb]).wait()

        a, b = abuf[pb], bbuf[pb]
        cvec = [col_v[h, pl.ds(16 * k, 16)] for k in range(8)]

        @plsc.parallel_loop(0, EMBED_DIM, unroll=4)
        def _tc(c):
            i = lax.shift_right_logical(c, 3)
            s = lax.rem(c, 8)
            cc = jnp.full((16,), c, jnp.int32)
            for k in range(8):
                b[i, s, pl.ds(16 * k, 16)] = plsc.load_gather(
                    a, [iotas[k], cc + cvec[k]])

        pltpu.async_copy(b, out_hbm.at[h, :, wid], osem[pb])

    @pl.loop(0, HIST, step=2)
    def _run(h0):
        step(h0, 0)
        step(h0 + 1, 1)

    for pb in range(2):
        pltpu.make_async_copy(bbuf[pb], out_hbm.at[HIST - 2 + pb, :, wid],
                              osem[pb]).wait()


def kernel(input, table):
    per_w = input.astype(jnp.int32).T.reshape(HIST, NW, BBLK).transpose(1, 0, 2)
    pad = jnp.zeros((NW, HPAD - HIST, BBLK), jnp.int32)
    idx_half = jnp.concatenate([per_w >> 1, pad], axis=1)
    colbase = jnp.concatenate([(per_w & 1) << 6, pad], axis=1)
    table2 = _pack_table(table.T)
    out5 = _embed_gather(idx_half, colbase, table2)
    return out5.transpose(2, 4, 0, 1, 3).reshape(BATCH, HIST, EMBED_DIM)
